# Initial kernel scaffold; baseline (speedup 1.0000x reference)
import jax
import jax.numpy as jnp
from jax.experimental import pallas as pl


def kernel(subtokens, table):
    raise NotImplementedError("write your pallas kernel here")


# speedup vs baseline: 4.6692x; 4.6692x over previous
import functools

import jax
import jax.numpy as jnp
from jax import lax
from jax.experimental import pallas as pl
from jax.experimental.pallas import tpu as pltpu
from jax.experimental.pallas import tpu_sc as plsc

N = 100000
L = 8
D = 64
C = 128
LANES = 16

_info = plsc.get_sparse_core_info()
NC, NS = _info.num_cores, _info.num_subcores
NW = NC * NS
FULL_CHUNKS = N // C
TAIL = N - FULL_CHUNKS * C
ROUNDS = -(-FULL_CHUNKS // NW)


@functools.partial(
    pl.kernel,
    mesh=plsc.VectorSubcoreMesh(core_axis_name="c", subcore_axis_name="s"),
    out_type=jax.ShapeDtypeStruct((N, D), jnp.float32),
    compiler_params=pltpu.CompilerParams(use_tc_tiling_on_sc=False),
    scratch_types=[
        pltpu.VMEM((L, C), jnp.int32),
        pltpu.VMEM((C * L, D), jnp.float32),
        pltpu.VMEM((C, D), jnp.float32),
        pltpu.SemaphoreType.DMA,
    ],
)
def _sc_embed(subtok_hbm, table_hbm, out_hbm, idx_v, rows_v, out_v, sem):
    wid = lax.axis_index("s") * NC + lax.axis_index("c")

    def pool_nodes(n_nodes):
        def node_body(i, _):
            b = i * L
            for c4 in range(D // LANES):
                acc = rows_v[b, pl.ds(c4 * LANES, LANES)]
                for j in range(1, L):
                    acc = acc + rows_v[b + j, pl.ds(c4 * LANES, LANES)]
                out_v[i, pl.ds(c4 * LANES, LANES)] = acc
            return 0

        lax.fori_loop(0, n_nodes, node_body, 0)

    def round_body(k, carry):
        g = k * NW + wid

        @pl.when(g < FULL_CHUNKS)
        def _():
            nb = pl.multiple_of(g * C, C)
            row = pl.multiple_of(g * L, L)
            pltpu.sync_copy(subtok_hbm.at[pl.ds(row, L)], idx_v)
            cps = [
                pltpu.async_copy(
                    table_hbm.at[idx_v.at[j]], rows_v.at[pl.ds(j * C, C)], sem
                )
                for j in range(L)
            ]
            for cp in cps:
                cp.wait()
            pool_nodes(C)
            pltpu.sync_copy(out_v, out_hbm.at[pl.ds(nb, C)])

        return carry

    lax.fori_loop(0, ROUNDS, round_body, 0)

    TROWS = TAIL * L // C

    @pl.when(wid == NW - 1)
    def _tail():
        pltpu.sync_copy(
            subtok_hbm.at[pl.ds(FULL_CHUNKS * L, TROWS)], idx_v.at[pl.ds(0, TROWS)]
        )
        cps = [
            pltpu.async_copy(
                table_hbm.at[idx_v.at[j]], rows_v.at[pl.ds(j * C, C)], sem
            )
            for j in range(TROWS)
        ]
        for cp in cps:
            cp.wait()
        pool_nodes(TAIL)
        pltpu.sync_copy(
            out_v.at[pl.ds(0, TAIL)], out_hbm.at[pl.ds(FULL_CHUNKS * C, TAIL)]
        )


def kernel(subtokens, table):
    idx = jnp.reshape(subtokens.astype(jnp.int32), (N * L // C, C))
    return _sc_embed(idx, table)


# speedup vs baseline: 5.7729x; 1.2364x over previous
import functools

import jax
import jax.numpy as jnp
from jax import lax
from jax.experimental import pallas as pl
from jax.experimental.pallas import tpu as pltpu
from jax.experimental.pallas import tpu_sc as plsc

N = 100000
L = 8
D = 64
C = 128
H = C // 2
LANES = 16

_info = plsc.get_sparse_core_info()
NC, NS = _info.num_cores, _info.num_subcores
NW = NC * NS
FULL_CHUNKS = N // C
TAIL = N - FULL_CHUNKS * C
ROUNDS = -(-FULL_CHUNKS // NW)


@functools.partial(
    pl.kernel,
    mesh=plsc.VectorSubcoreMesh(core_axis_name="c", subcore_axis_name="s"),
    out_type=jax.ShapeDtypeStruct((N, D), jnp.float32),
    compiler_params=pltpu.CompilerParams(use_tc_tiling_on_sc=False),
    scratch_types=[
        pltpu.VMEM((2, L, C), jnp.int32),
        pltpu.VMEM((C * L, D), jnp.float32),
        pltpu.VMEM((2, C, D), jnp.float32),
        pltpu.SemaphoreType.DMA((2,)),
        pltpu.SemaphoreType.DMA((2,)),
        pltpu.SemaphoreType.DMA,
    ],
)
def _sc_embed(subtok_hbm, table_hbm, out_hbm, idx_v, rows_v, out_v, gsem, osem, isem):
    wid = lax.axis_index("s") * NC + lax.axis_index("c")

    def chunk_of(k):
        return jnp.minimum(k * NW + wid, FULL_CHUNKS - 1)

    def load_ids(k, sync):
        g = chunk_of(k)
        row = pl.multiple_of(g * L, L)
        src = subtok_hbm.at[pl.ds(row, L)]
        if sync:
            pltpu.sync_copy(src, idx_v.at[k % 2])
            return None
        return pltpu.async_copy(src, idx_v.at[k % 2], isem)

    def issue_half(k, h):
        return [
            pltpu.async_copy(
                table_hbm.at[idx_v.at[k % 2, h * 4 + j]],
                rows_v.at[pl.ds((h * 4 + j) * C, C)],
                gsem.at[h],
            )
            for j in range(4)
        ]

    def pool(hb, n_nodes, p):
        def node_body(i, _):
            b = (hb + i) * L
            for c4 in range(D // LANES):
                acc = rows_v[b, pl.ds(c4 * LANES, LANES)]
                for j in range(1, L):
                    acc = acc + rows_v[b + j, pl.ds(c4 * LANES, LANES)]
                out_v[p, hb + i, pl.ds(c4 * LANES, LANES)] = acc
            return 0

        lax.fori_loop(0, n_nodes, node_body, 0)

    load_ids(0, sync=True)
    halves = [issue_half(0, 0), issue_half(0, 1)]
    ocps = {}

    for k in range(ROUNDS):
        p = k % 2
        icp = load_ids(k + 1, sync=False) if k + 1 < ROUNDS else None
        if k >= 2:
            ocps[k - 2].wait()
        for cp in halves[0]:
            cp.wait()
        pool(0, H, p)
        if icp is not None:
            icp.wait()
            halves[0] = issue_half(k + 1, 0)
        for cp in halves[1]:
            cp.wait()
        pool(H, H, p)
        if k + 1 < ROUNDS:
            halves[1] = issue_half(k + 1, 1)
        nb = pl.multiple_of(chunk_of(k) * C, C)
        ocps[k] = pltpu.async_copy(out_v.at[p], out_hbm.at[pl.ds(nb, C)], osem.at[p])

    ocps[ROUNDS - 2].wait()
    ocps[ROUNDS - 1].wait()

    TROWS = TAIL * L // C

    @pl.when(wid == NW - 1)
    def _tail():
        pltpu.sync_copy(
            subtok_hbm.at[pl.ds(FULL_CHUNKS * L, TROWS)],
            idx_v.at[0, pl.ds(0, TROWS)],
        )
        tcps = [
            pltpu.async_copy(
                table_hbm.at[idx_v.at[0, j]],
                rows_v.at[pl.ds(j * C, C)],
                gsem.at[0],
            )
            for j in range(TROWS)
        ]
        for cp in tcps:
            cp.wait()
        pool(0, TAIL, 0)
        pltpu.sync_copy(
            out_v.at[0, pl.ds(0, TAIL)], out_hbm.at[pl.ds(FULL_CHUNKS * C, TAIL)]
        )


def kernel(subtokens, table):
    idx = jnp.reshape(subtokens.astype(jnp.int32), (N * L // C, C))
    return _sc_embed(idx, table)


# speedup vs baseline: 8.6988x; 1.5068x over previous
import functools

import jax
import jax.numpy as jnp
from jax import lax
from jax.experimental import pallas as pl
from jax.experimental.pallas import tpu as pltpu
from jax.experimental.pallas import tpu_sc as plsc

N = 100000
L = 8
D = 64
C = 128
LANES = 16

_info = plsc.get_sparse_core_info()
NC, NS = _info.num_cores, _info.num_subcores
NW = NC * NS
FULL_CHUNKS = N // C
TAIL = N - FULL_CHUNKS * C
ROUNDS = -(-FULL_CHUNKS // NW)


@functools.partial(
    pl.kernel,
    mesh=plsc.VectorSubcoreMesh(core_axis_name="c", subcore_axis_name="s"),
    out_type=jax.ShapeDtypeStruct((N, D), jnp.float32),
    compiler_params=pltpu.CompilerParams(use_tc_tiling_on_sc=False),
    scratch_types=[
        pltpu.VMEM((2, L, C), jnp.int32),
        pltpu.VMEM((L * C, D), jnp.float32),
        pltpu.VMEM((2, C, D), jnp.float32),
        pltpu.SemaphoreType.DMA((2,)),
        pltpu.SemaphoreType.DMA((2,)),
        pltpu.SemaphoreType.DMA,
    ],
)
def _sc_embed(subtok_hbm, table_hbm, out_hbm, idx_v, rows_v, out_v, gsem, osem, isem):
    wid = lax.axis_index("s") * NC + lax.axis_index("c")

    def chunk_of(k):
        return jnp.minimum(k * NW + wid, FULL_CHUNKS - 1)

    def load_ids(k):
        nb = pl.multiple_of(chunk_of(k) * C, C)
        return pltpu.async_copy(
            subtok_hbm.at[:, pl.ds(nb, C)], idx_v.at[k % 2], isem
        )

    def issue_half(k, h):
        return [
            pltpu.async_copy(
                table_hbm.at[idx_v.at[k % 2, 4 * h + dl]],
                rows_v.at[pl.ds((4 * h + dl) * C, C)],
                gsem.at[h],
            )
            for dl in range(4)
        ]

    def pool(n_nodes, p, stride, ls, accumulate):
        def node_body(m, _):
            for c4 in range(D // LANES):
                acc = rows_v[ls[0] * stride + m, pl.ds(c4 * LANES, LANES)]
                for l in ls[1:]:
                    acc = acc + rows_v[l * stride + m, pl.ds(c4 * LANES, LANES)]
                if accumulate:
                    plsc.addupdate(out_v.at[p, m, pl.ds(c4 * LANES, LANES)], acc)
                else:
                    out_v[p, m, pl.ds(c4 * LANES, LANES)] = acc
            return 0

        lax.fori_loop(0, n_nodes, node_body, 0)

    load_ids(0).wait()
    halves = [issue_half(0, 0), issue_half(0, 1)]
    ocps = {}

    for k in range(ROUNDS):
        p = k % 2
        icp = load_ids(k + 1) if k + 1 < ROUNDS else None
        if k >= 2:
            ocps[k - 2].wait()
        for cp in halves[0]:
            cp.wait()
        pool(C, p, C, [0, 1, 2, 3], accumulate=False)
        if icp is not None:
            icp.wait()
            halves[0] = issue_half(k + 1, 0)
        for cp in halves[1]:
            cp.wait()
        pool(C, p, C, [4, 5, 6, 7], accumulate=True)
        if k + 1 < ROUNDS:
            halves[1] = issue_half(k + 1, 1)
        nb = pl.multiple_of(chunk_of(k) * C, C)
        ocps[k] = pltpu.async_copy(out_v.at[p], out_hbm.at[pl.ds(nb, C)], osem.at[p])

    ocps[ROUNDS - 2].wait()
    ocps[ROUNDS - 1].wait()

    @pl.when(wid == NW - 1)
    def _tail():
        pltpu.sync_copy(
            subtok_hbm.at[:, pl.ds(FULL_CHUNKS * C, TAIL)],
            idx_v.at[0, :, pl.ds(0, TAIL)],
        )
        tcps = [
            pltpu.async_copy(
                table_hbm.at[idx_v.at[0, l, pl.ds(0, TAIL)]],
                rows_v.at[pl.ds(l * TAIL, TAIL)],
                gsem.at[0],
            )
            for l in range(L)
        ]
        for cp in tcps:
            cp.wait()
        pool(TAIL, 0, TAIL, list(range(L)), accumulate=False)
        pltpu.sync_copy(
            out_v.at[0, pl.ds(0, TAIL)], out_hbm.at[pl.ds(FULL_CHUNKS * C, TAIL)]
        )


def kernel(subtokens, table):
    ids = jnp.transpose(subtokens.astype(jnp.int32))
    return _sc_embed(ids, table)
